# Initial kernel scaffold; baseline (speedup 1.0000x reference)
import jax
import jax.numpy as jnp
from jax.experimental import pallas as pl


def kernel(x_batch, LOS_batch, ad_idx_t, dis_idx_t, tables, W_ih, W_hh, b_ih, b_hh, W1, b1, W2, b2):
    raise NotImplementedError("write your pallas kernel here")


# speedup vs baseline: 2.8138x; 2.8138x over previous
import functools

import jax
import jax.numpy as jnp
from jax import lax
from jax.experimental import pallas as pl
from jax.experimental.pallas import tpu as pltpu
from jax.experimental.pallas import tpu_sc as plsc

_B = 16384
_NUM_VAR = 26
_VOCAB = 100000
_D = 32
_H = 64
_MAX_LOS = 37

_NC = 2
_NS = 16
_NW = _NC * _NS
_BPW = _B // _NW
_CB = 64


def _sc_gather_sums(flat_idx, table_flat):
    mesh = plsc.VectorSubcoreMesh(core_axis_name="c", subcore_axis_name="s")
    n_chunks = _BPW // _CB

    @functools.partial(
        pl.kernel,
        mesh=mesh,
        compiler_params=pltpu.CompilerParams(use_tc_tiling_on_sc=False),
        out_type=(
            jax.ShapeDtypeStruct((_B, _D), jnp.float32),
            jax.ShapeDtypeStruct((_B, _D), jnp.float32),
        ),
        scratch_types=[
            pltpu.VMEM((_CB * _NUM_VAR,), jnp.int32),
            pltpu.VMEM((_CB * _NUM_VAR, _D), jnp.float32),
            pltpu.VMEM((_CB, _D), jnp.float32),
            pltpu.VMEM((_CB, _D), jnp.float32),
            pltpu.SemaphoreType.DMA,
        ],
    )
    def k(idx_hbm, tab_hbm, ad_hbm, dis_hbm, idx_v, rows_v, ad_v, dis_v, sem):
        wid = lax.axis_index("s") * _NC + lax.axis_index("c")
        for ci in range(n_chunks):
            base_b = wid * _BPW + ci * _CB
            pltpu.sync_copy(
                idx_hbm.at[pl.ds(base_b * _NUM_VAR, _CB * _NUM_VAR)], idx_v)
            pltpu.async_copy(tab_hbm.at[idx_v], rows_v, sem).wait()

            def body(b, carry):
                row = b * _NUM_VAR
                a0 = rows_v[row, 0:16]
                a1 = rows_v[row, 16:32]
                for j in range(1, 13):
                    a0 = a0 + rows_v[row + j, 0:16]
                    a1 = a1 + rows_v[row + j, 16:32]
                d0 = rows_v[row + 13, 0:16]
                d1 = rows_v[row + 13, 16:32]
                for j in range(14, 26):
                    d0 = d0 + rows_v[row + j, 0:16]
                    d1 = d1 + rows_v[row + j, 16:32]
                ad_v[b, 0:16] = a0
                ad_v[b, 16:32] = a1
                dis_v[b, 0:16] = d0
                dis_v[b, 16:32] = d1
                return carry

            lax.fori_loop(0, _CB, body, 0)
            pltpu.sync_copy(ad_v, ad_hbm.at[pl.ds(base_b, _CB)])
            pltpu.sync_copy(dis_v, dis_hbm.at[pl.ds(base_b, _CB)])

    return k(flat_idx, table_flat)


def _gru_head(ad, dis, los, w_ih_t, b_ih, w_hh_t, b_hh, w1_t, b1, w2_t, b2):
    bb = 1024
    grid = (_B // bb,)

    def body(ad_ref, dis_ref, los_ref, wih_ref, bih_ref, whh_ref, bhh_ref,
             w1_ref, b1_ref, w2_ref, b2_ref, out_ref):
        l = los_ref[...]
        wih = wih_ref[...]
        bih = bih_ref[...]
        gi_ad = jnp.dot(ad_ref[...], wih,
                        preferred_element_type=jnp.float32) + bih
        gi_dis = jnp.dot(dis_ref[...], wih,
                         preferred_element_type=jnp.float32) + bih
        whh = whh_ref[...]
        bhh = bhh_ref[...]

        def step(t, h):
            gh = jnp.dot(h, whh, preferred_element_type=jnp.float32) + bhh
            gi = jnp.where(l == t + 1, gi_dis, gi_ad)
            r = jax.nn.sigmoid(gi[:, 0:_H] + gh[:, 0:_H])
            z = jax.nn.sigmoid(gi[:, _H:2 * _H] + gh[:, _H:2 * _H])
            n = jnp.tanh(gi[:, 2 * _H:] + r * gh[:, 2 * _H:])
            h_new = (1.0 - z) * n + z * h
            return jnp.where(l > t, h_new, h)

        h = lax.fori_loop(0, _MAX_LOS, step,
                          jnp.zeros((bb, _H), jnp.float32))
        hid = jnp.maximum(
            jnp.dot(h, w1_ref[...], preferred_element_type=jnp.float32)
            + b1_ref[...], 0.0)
        out_ref[...] = (jnp.dot(hid, w2_ref[...],
                                preferred_element_type=jnp.float32)
                        + b2_ref[...])

    full = lambda shape: pl.BlockSpec(shape, lambda i: (0,) * len(shape))
    return pl.pallas_call(
        body,
        grid=grid,
        in_specs=[
            pl.BlockSpec((bb, _D), lambda i: (i, 0)),
            pl.BlockSpec((bb, _D), lambda i: (i, 0)),
            pl.BlockSpec((bb, 1), lambda i: (i, 0)),
            full((_D, 3 * _H)),
            full((1, 3 * _H)),
            full((_H, 3 * _H)),
            full((1, 3 * _H)),
            full((_H, 128)),
            full((1, 128)),
            full((128, 1)),
            full((1, 1)),
        ],
        out_specs=pl.BlockSpec((bb, 1), lambda i: (i, 0)),
        out_shape=jax.ShapeDtypeStruct((_B, 1), jnp.float32),
    )(ad, dis, los, w_ih_t, b_ih, w_hh_t, b_hh, w1_t, b1, w2_t, b2)


def kernel(x_batch, LOS_batch, ad_idx_t, dis_idx_t, tables,
           W_ih, W_hh, b_ih, b_hh, W1, b1, W2, b2):
    x = x_batch.astype(jnp.int32)
    col_off = (jnp.arange(_NUM_VAR, dtype=jnp.int32) * _VOCAB)[None, :]
    flat_idx = (x + col_off).reshape(-1)
    table_flat = tables.reshape(_NUM_VAR * _VOCAB, _D)
    ad, dis = _sc_gather_sums(flat_idx, table_flat)
    los = LOS_batch.astype(jnp.int32).reshape(_B, 1)
    return _gru_head(
        ad, dis, los,
        W_ih.T, b_ih.reshape(1, -1),
        W_hh.T, b_hh.reshape(1, -1),
        W1.T, b1.reshape(1, -1),
        W2.T, b2.reshape(1, 1))
